# speedup vs baseline: 1.0672x; 1.0672x over previous
import jax
import jax.numpy as jnp
from jax.experimental import pallas as pl


def _ln(h, g, b):
    mu = jnp.mean(h, axis=-1, keepdims=True)
    var = jnp.var(h, axis=-1, keepdims=True)
    h = (h - mu) * jax.lax.rsqrt(var + 1e-5)
    return h * g + b


def kernel(x, edge_attr, edge_index, params, batch_size, shard_shapes):
    N, C = x.shape
    src = edge_index[0]
    dst = edge_index[1]
    h = x
    e = edge_attr
    for p in params:
        em = p["edge_mlp"]
        W1 = em["W1"]
        W1d, W1s, W1e = W1[:C], W1[C:2 * C], W1[2 * C:]
        A = jnp.dot(h, W1d) + em["b1"]
        B = jnp.dot(h, W1s)
        Epre = jnp.dot(e, W1e)
        hid = jax.nn.silu(A[dst] + B[src] + Epre)
        e_new = _ln(jnp.dot(hid, em["W2"]) + em["b2"], em["ln_g"], em["ln_b"])
        if e.shape[-1] == e_new.shape[-1]:
            e_new = e_new + e
        agg = jax.ops.segment_sum(e_new, dst, num_segments=N)
        nm = p["node_mlp"]
        Wh, Wa = nm["W1"][:C], nm["W1"][C:]
        hid2 = jax.nn.silu(jnp.dot(h, Wh) + jnp.dot(agg, Wa) + nm["b1"])
        h = _ln(jnp.dot(hid2, nm["W2"]) + nm["b2"], nm["ln_g"], nm["ln_b"]) + h
        e = e_new
    return h


# speedup vs baseline: 3.2243x; 3.0212x over previous
import functools

import jax
import jax.numpy as jnp
from jax import lax
from jax.experimental import pallas as pl
from jax.experimental.pallas import tpu as pltpu
from jax.experimental.pallas import tpu_sc as plsc

F32 = jnp.float32
BN = 1000
BE = 2000
KC = 200


def _ln(h, g, b):
    mu = jnp.mean(h, axis=-1, keepdims=True)
    var = jnp.mean((h - mu) ** 2, axis=-1, keepdims=True)
    return (h - mu) * lax.rsqrt(var + 1e-5) * g + b


def _silu(x):
    return x * lax.logistic(x)


def _ab_body(h_ref, wd_ref, ws_ref, b1_ref, a_ref, b_ref):
    h = h_ref[...]
    a_ref[...] = jnp.dot(h, wd_ref[...], preferred_element_type=F32) + b1_ref[...]
    b_ref[...] = jnp.dot(h, ws_ref[...], preferred_element_type=F32)


def _ab(h, wd, ws, b1):
    N, C = h.shape
    return pl.pallas_call(
        _ab_body,
        grid=(N // BN,),
        in_specs=[
            pl.BlockSpec((BN, C), lambda i: (i, 0)),
            pl.BlockSpec((C, C), lambda i: (0, 0)),
            pl.BlockSpec((C, C), lambda i: (0, 0)),
            pl.BlockSpec((1, C), lambda i: (0, 0)),
        ],
        out_specs=[pl.BlockSpec((BN, C), lambda i: (i, 0)),
                   pl.BlockSpec((BN, C), lambda i: (i, 0))],
        out_shape=[jax.ShapeDtypeStruct((N, C), F32)] * 2,
    )(h, wd, ws, b1.reshape(1, C))


def _epre_body(e_ref, w_ref, o_ref):
    o_ref[...] = jnp.dot(e_ref[...], w_ref[...], preferred_element_type=F32)


def _epre(e, w):
    E, D = e.shape
    C = w.shape[1]
    return pl.pallas_call(
        _epre_body,
        grid=(E // BE,),
        in_specs=[pl.BlockSpec((BE, D), lambda i: (i, 0)),
                  pl.BlockSpec((D, C), lambda i: (0, 0))],
        out_specs=pl.BlockSpec((BE, C), lambda i: (i, 0)),
        out_shape=jax.ShapeDtypeStruct((E, C), F32),
    )(e, w)


def _edge_body_res(ga_ref, gb_ref, ep_ref, ev_ref, w2_ref, b2_ref, g_ref,
                   bl_ref, o_ref):
    hid = _silu(ga_ref[...] + gb_ref[...] + ep_ref[...])
    out = jnp.dot(hid, w2_ref[...], preferred_element_type=F32) + b2_ref[...]
    o_ref[...] = _ln(out, g_ref[...], bl_ref[...]) + ev_ref[...]


def _edge_body_nores(ga_ref, gb_ref, ep_ref, w2_ref, b2_ref, g_ref,
                     bl_ref, o_ref):
    hid = _silu(ga_ref[...] + gb_ref[...] + ep_ref[...])
    out = jnp.dot(hid, w2_ref[...], preferred_element_type=F32) + b2_ref[...]
    o_ref[...] = _ln(out, g_ref[...], bl_ref[...])


def _edge(ga, gb, ep, em, e_prev):
    E, C = ga.shape
    blk = lambda: pl.BlockSpec((BE, C), lambda i: (i, 0))
    wspec = [pl.BlockSpec((C, C), lambda i: (0, 0))] + \
            [pl.BlockSpec((1, C), lambda i: (0, 0))] * 3
    wargs = (em["W2"], em["b2"].reshape(1, C), em["ln_g"].reshape(1, C),
             em["ln_b"].reshape(1, C))
    if e_prev is not None:
        return pl.pallas_call(
            _edge_body_res,
            grid=(E // BE,),
            in_specs=[blk(), blk(), blk(), blk()] + wspec,
            out_specs=blk(),
            out_shape=jax.ShapeDtypeStruct((E, C), F32),
        )(ga, gb, ep, e_prev, *wargs)
    return pl.pallas_call(
        _edge_body_nores,
        grid=(E // BE,),
        in_specs=[blk(), blk(), blk()] + wspec,
        out_specs=blk(),
        out_shape=jax.ShapeDtypeStruct((E, C), F32),
    )(ga, gb, ep, *wargs)


def _node_body(h_ref, p0_ref, p1_ref, wh_ref, wa_ref, b1_ref, w2_ref, b2_ref,
               g_ref, bl_ref, ho_ref):
    h = h_ref[...]
    agg = p0_ref[...] + p1_ref[...]
    hid = _silu(jnp.dot(h, wh_ref[...], preferred_element_type=F32)
                + jnp.dot(agg, wa_ref[...], preferred_element_type=F32)
                + b1_ref[...])
    out = jnp.dot(hid, w2_ref[...], preferred_element_type=F32) + b2_ref[...]
    ho_ref[...] = _ln(out, g_ref[...], bl_ref[...]) + h


def _node_body_ab(h_ref, p0_ref, p1_ref, wh_ref, wa_ref, b1_ref, w2_ref,
                  b2_ref, g_ref, bl_ref, wdn_ref, wsn_ref, b1n_ref,
                  ho_ref, a_ref, b_ref):
    h = h_ref[...]
    agg = p0_ref[...] + p1_ref[...]
    hid = _silu(jnp.dot(h, wh_ref[...], preferred_element_type=F32)
                + jnp.dot(agg, wa_ref[...], preferred_element_type=F32)
                + b1_ref[...])
    out = jnp.dot(hid, w2_ref[...], preferred_element_type=F32) + b2_ref[...]
    hn = _ln(out, g_ref[...], bl_ref[...]) + h
    ho_ref[...] = hn
    a_ref[...] = jnp.dot(hn, wdn_ref[...], preferred_element_type=F32) + b1n_ref[...]
    b_ref[...] = jnp.dot(hn, wsn_ref[...], preferred_element_type=F32)


def _node(h, p0, p1, nm, next_em):
    N, C = h.shape
    blk = lambda: pl.BlockSpec((BN, C), lambda i: (i, 0))
    cc = lambda: pl.BlockSpec((C, C), lambda i: (0, 0))
    rc = lambda: pl.BlockSpec((1, C), lambda i: (0, 0))
    W1 = nm["W1"]
    wargs = (W1[:C], W1[C:], nm["b1"].reshape(1, C), nm["W2"],
             nm["b2"].reshape(1, C), nm["ln_g"].reshape(1, C),
             nm["ln_b"].reshape(1, C))
    if next_em is None:
        return pl.pallas_call(
            _node_body,
            grid=(N // BN,),
            in_specs=[blk(), blk(), blk(), cc(), cc(), rc(), cc(), rc(),
                      rc(), rc()],
            out_specs=blk(),
            out_shape=jax.ShapeDtypeStruct((N, C), F32),
        )(h, p0, p1, *wargs)
    nW1 = next_em["W1"]
    return pl.pallas_call(
        _node_body_ab,
        grid=(N // BN,),
        in_specs=[blk(), blk(), blk(), cc(), cc(), rc(), cc(), rc(),
                  rc(), rc(), cc(), cc(), rc()],
        out_specs=[blk(), blk(), blk()],
        out_shape=[jax.ShapeDtypeStruct((N, C), F32)] * 3,
    )(h, p0, p1, *wargs, nW1[:C], nW1[C:2 * C], next_em["b1"].reshape(1, C))


def _make_sc_fns(N, C, E):
    info = plsc.get_sparse_core_info()
    ncore, nsub = info.num_cores, info.num_subcores
    nw = ncore * nsub
    epw = E // nw
    assert E % nw == 0 and epw % KC == 0 and N % nsub == 0
    rpt = N // nsub
    mesh = plsc.VectorSubcoreMesh(core_axis_name="c", subcore_axis_name="s")

    @functools.partial(
        pl.kernel,
        out_type=(jax.ShapeDtypeStruct((E, C), F32),
                  jax.ShapeDtypeStruct((E, C), F32)),
        mesh=mesh,
        scratch_types=[
            pltpu.VMEM((KC,), jnp.int32),
            pltpu.VMEM((KC,), jnp.int32),
            pltpu.VMEM((KC, C), F32),
            pltpu.VMEM((KC, C), F32),
            pltpu.SemaphoreType.DMA,
            pltpu.SemaphoreType.DMA,
        ],
    )
    def gather(a_hbm, b_hbm, dst_hbm, src_hbm, oa_hbm, ob_hbm,
               di, si, ra, rb, s1, s2):
        wid = lax.axis_index("c") * nsub + lax.axis_index("s")
        base = wid * epw

        @pl.loop(0, epw // KC)
        def _(k):
            off = base + k * KC
            pltpu.sync_copy(dst_hbm.at[pl.ds(off, KC)], di)
            pltpu.sync_copy(src_hbm.at[pl.ds(off, KC)], si)
            cpa = pltpu.async_copy(a_hbm.at[di], ra, s1)
            cpb = pltpu.async_copy(b_hbm.at[si], rb, s2)
            cpa.wait()
            cpb.wait()
            pltpu.sync_copy(ra, oa_hbm.at[pl.ds(off, KC)])
            pltpu.sync_copy(rb, ob_hbm.at[pl.ds(off, KC)])

    @functools.partial(
        pl.kernel,
        out_type=jax.ShapeDtypeStruct((ncore, N, C), F32),
        mesh=mesh,
        scratch_types=[
            pltpu.VMEM_SHARED((N, C), F32),
            pltpu.VMEM((KC,), jnp.int32),
            pltpu.VMEM((KC, C), F32),
        ],
    )
    def scatter(e_hbm, dst_hbm, zero_hbm, o_hbm, acc, di, rows):
        c = lax.axis_index("c")
        s = lax.axis_index("s")
        base = (c * nsub + s) * epw

        @pl.when(s == 0)
        def _():
            pltpu.sync_copy(zero_hbm, acc)

        plsc.subcore_barrier()

        @pl.loop(0, epw // KC)
        def _(k):
            off = base + k * KC
            pltpu.sync_copy(dst_hbm.at[pl.ds(off, KC)], di)
            pltpu.sync_copy(e_hbm.at[pl.ds(off, KC)], rows)
            pltpu.sync_copy(rows, acc.at[di], add=True)

        plsc.subcore_barrier()

        @pl.when(s == 0)
        def _():
            pltpu.sync_copy(acc, o_hbm.at[c])

    return gather, scatter


def kernel(x, edge_attr, edge_index, params, batch_size, shard_shapes):
    N, C = x.shape
    E = edge_index.shape[1]
    src = edge_index[0]
    dst = edge_index[1]
    assert N % BN == 0 and E % BE == 0
    gather, scatter = _make_sc_fns(N, C, E)
    zeros_nc = jnp.zeros((N, C), F32)

    h = x
    e = edge_attr
    L = len(params)
    A = B = None
    epre = None
    for l, p in enumerate(params):
        em = p["edge_mlp"]
        W1 = em["W1"]
        if l == 0:
            A, B = _ab(h, W1[:C], W1[C:2 * C], em["b1"])
            epre = _epre(e, W1[2 * C:])
        ga, gb = gather(A, B, dst, src)
        e_new = _edge(ga, gb, epre, em, e if l > 0 else None)
        partials = scatter(e_new, dst, zeros_nc)
        if l + 1 < L:
            epre = _epre(e_new, params[l + 1]["edge_mlp"]["W1"][2 * C:])
            res = _node(h, partials[0], partials[1], p["node_mlp"],
                        params[l + 1]["edge_mlp"])
            h, A, B = res
        else:
            h = _node(h, partials[0], partials[1], p["node_mlp"], None)
        e = e_new
    return h
